# Initial kernel scaffold; baseline (speedup 1.0000x reference)
import jax
import jax.numpy as jnp
from jax.experimental import pallas as pl


def kernel(x, edge_index_r0, edge_index_r1, edge_index_r2, W_r0, b_r0, W_r1, b_r1, W_r2, b_r2):
    raise NotImplementedError("write your pallas kernel here")


# speedup vs baseline: 8.1469x; 8.1469x over previous
import functools

import jax
import jax.numpy as jnp
from jax import lax
from jax.experimental import pallas as pl
from jax.experimental.pallas import tpu as pltpu
from jax.experimental.pallas import tpu_sc as plsc

N = 10000
D = 128
E = 320000
NREL = 3
NC = 2
NS = 16
NW = NC * NS

HIST = 10240
HROWS = HIST // NS

E_PER_CORE = E // NC
E_PER_TILE = E // NW
DEG_CHUNK = 2000
DEG_CHUNKS = E_PER_TILE // DEG_CHUNK

AGG_CHUNK = 200
AGG_CHUNKS = E_PER_TILE // AGG_CHUNK
N_PAD = 10240
ROWS_PER_TILE = N_PAD // NS
DUMP_CHUNK = 64
DUMP_CHUNKS = ROWS_PER_TILE // DUMP_CHUNK

_mesh = plsc.VectorSubcoreMesh(core_axis_name="c", subcore_axis_name="s")


def _deg_body(s0, d0, s1, d1, s2, d2, ones, zeros, out_hbm,
              h0_sh, h1_sh, h2_sh, h3_sh, h4_sh, h5_sh,
              idx_v, ones_v, zero_v, bounce_v):
  c = lax.axis_index("c")
  s = lax.axis_index("s")
  hists = (h0_sh, h1_sh, h2_sh, h3_sh, h4_sh, h5_sh)
  pltpu.sync_copy(zeros, zero_v)
  pltpu.sync_copy(ones, ones_v)
  for j in range(6):
    pltpu.sync_copy(zero_v, hists[j].at[pl.ds(s * HROWS, HROWS)])
  plsc.subcore_barrier()
  eis = (s0, d0, s1, d1, s2, d2)
  for j in range(6):
    for k in range(DEG_CHUNKS):
      off = c * E_PER_CORE + s * E_PER_TILE + k * DEG_CHUNK
      pltpu.sync_copy(eis[j].at[pl.ds(off, DEG_CHUNK)], idx_v)
      pltpu.sync_copy(ones_v, hists[j].at[idx_v], add=True)
  plsc.subcore_barrier()
  for j in range(6):
    pltpu.sync_copy(hists[j].at[pl.ds(s * HROWS, HROWS)], bounce_v)
    pltpu.sync_copy(bounce_v,
                    out_hbm.at[c, 0, pl.ds(j * HIST + s * HROWS, HROWS)])


_deg_kernel = pl.kernel(
    _deg_body,
    out_type=jax.ShapeDtypeStruct((NC, 1, 6 * HIST), jnp.float32),
    mesh=_mesh,
    scratch_types=[
        pltpu.VMEM_SHARED((HIST,), jnp.float32),
        pltpu.VMEM_SHARED((HIST,), jnp.float32),
        pltpu.VMEM_SHARED((HIST,), jnp.float32),
        pltpu.VMEM_SHARED((HIST,), jnp.float32),
        pltpu.VMEM_SHARED((HIST,), jnp.float32),
        pltpu.VMEM_SHARED((HIST,), jnp.float32),
        pltpu.VMEM((DEG_CHUNK,), jnp.int32),
        pltpu.VMEM((DEG_CHUNK,), jnp.float32),
        pltpu.VMEM((HROWS,), jnp.float32),
        pltpu.VMEM((HROWS,), jnp.float32),
    ],
)


def _agg_body(h0, h1, h2, s0, d0, s1, d1, s2, d2, zeros, out0, out1, out2,
              agg_sh, sidx_v, didx_v, rows_v, zb_v):
  c = lax.axis_index("c")
  s = lax.axis_index("s")
  hs = (h0, h1, h2)
  srcs = (s0, s1, s2)
  dsts = (d0, d1, d2)
  outs = (out0, out1, out2)
  base = c * E_PER_CORE + s * E_PER_TILE
  for r in range(NREL):
    pltpu.sync_copy(zeros, zb_v)
    for z in range(DUMP_CHUNKS):
      pltpu.sync_copy(
          zb_v, agg_sh.at[pl.ds(s * ROWS_PER_TILE + z * DUMP_CHUNK,
                                DUMP_CHUNK)])
    plsc.subcore_barrier()

    def chunk(k, carry, r=r):
      off = pl.multiple_of(base + k * AGG_CHUNK, 8)
      pltpu.sync_copy(srcs[r].at[pl.ds(off, AGG_CHUNK)], sidx_v)
      pltpu.sync_copy(dsts[r].at[pl.ds(off, AGG_CHUNK)], didx_v)
      pltpu.sync_copy(hs[r].at[sidx_v], rows_v)
      pltpu.sync_copy(rows_v, agg_sh.at[didx_v], add=True)
      return carry

    lax.fori_loop(0, AGG_CHUNKS, chunk, 0)
    plsc.subcore_barrier()
    for z in range(DUMP_CHUNKS):
      row0 = s * ROWS_PER_TILE + z * DUMP_CHUNK
      pltpu.sync_copy(agg_sh.at[pl.ds(row0, DUMP_CHUNK)], zb_v)
      pltpu.sync_copy(zb_v, outs[r].at[c, pl.ds(row0, DUMP_CHUNK)])
    plsc.subcore_barrier()


_agg_kernel = pl.kernel(
    _agg_body,
    out_type=(jax.ShapeDtypeStruct((NC, N_PAD, D), jnp.float32),) * NREL,
    mesh=_mesh,
    scratch_types=[
        pltpu.VMEM_SHARED((N_PAD, D), jnp.float32),
        pltpu.VMEM((AGG_CHUNK,), jnp.int32),
        pltpu.VMEM((AGG_CHUNK,), jnp.int32),
        pltpu.VMEM((AGG_CHUNK, D), jnp.float32),
        pltpu.VMEM((DUMP_CHUNK, D), jnp.float32),
    ],
)


def _norm(deg):
  return jnp.where(deg > 0, lax.rsqrt(jnp.maximum(deg, 1e-12)), 0.0)


def _h_body(x_ref, degs_ref, h_ref):
  for r in range(NREL):
    deg = degs_ref[0, :N, 2 * r:2 * r + 1] + degs_ref[1, :N, 2 * r:2 * r + 1]
    h_ref[r] = x_ref[...] * _norm(deg)


def _h_kernel(x, degs):
  return pl.pallas_call(
      _h_body,
      out_shape=jax.ShapeDtypeStruct((NREL, N, D), jnp.float32),
  )(x, degs)


BLK = 1000


def _comb_body(degs_ref, p0_ref, p1_ref, p2_ref, w_ref, b_ref, out_ref):
  ps = (p0_ref, p1_ref, p2_ref)
  acc = jnp.zeros((BLK, D), jnp.float32)
  for r in range(NREL):
    deg = degs_ref[0, :, 2 * r + 1:2 * r + 2] + degs_ref[1, :, 2 * r + 1:2 * r + 2]
    agg = (ps[r][0] + ps[r][1]) * _norm(deg)
    acc += jnp.dot(agg, w_ref[r], preferred_element_type=jnp.float32)
  bsum = b_ref[0:1, :] + b_ref[1:2, :] + b_ref[2:3, :]
  out_ref[...] = acc * (1.0 / NREL) + bsum * (1.0 / NREL)


def _comb_kernel(degs, p0, p1, p2, W, B):
  grid = N // BLK
  part_spec = pl.BlockSpec((NC, BLK, D), lambda i: (0, i, 0))
  return pl.pallas_call(
      _comb_body,
      grid=(grid,),
      in_specs=[
          pl.BlockSpec((NC, BLK, 6), lambda i: (0, i, 0)),
          part_spec, part_spec, part_spec,
          pl.BlockSpec((NREL, D, D), lambda i: (0, 0, 0)),
          pl.BlockSpec((NREL, D), lambda i: (0, 0)),
      ],
      out_specs=pl.BlockSpec((BLK, D), lambda i: (i, 0)),
      out_shape=jax.ShapeDtypeStruct((N, D), jnp.float32),
  )(degs, p0, p1, p2, W, B)


@jax.jit
def kernel(x, edge_index_r0, edge_index_r1, edge_index_r2,
           W_r0, b_r0, W_r1, b_r1, W_r2, b_r2):
  ones = jnp.ones((DEG_CHUNK,), jnp.float32)
  zeros_h = jnp.zeros((HROWS,), jnp.float32)
  zeros_a = jnp.zeros((DUMP_CHUNK, D), jnp.float32)
  s0, d0 = edge_index_r0[0], edge_index_r0[1]
  s1, d1 = edge_index_r1[0], edge_index_r1[1]
  s2, d2 = edge_index_r2[0], edge_index_r2[1]
  degs6 = _deg_kernel(s0, d0, s1, d1, s2, d2, ones, zeros_h)
  degs = jnp.swapaxes(degs6.reshape(NC, 6, HIST), 1, 2)
  h = _h_kernel(x, degs)
  p0, p1, p2 = _agg_kernel(h[0], h[1], h[2], s0, d0, s1, d1, s2, d2,
                           zeros_a)
  W = jnp.stack([W_r0, W_r1, W_r2])
  B = jnp.stack([b_r0, b_r1, b_r2])
  return _comb_kernel(degs, p0, p1, p2, W, B)


# speedup vs baseline: 12.4031x; 1.5224x over previous
import functools

import jax
import jax.numpy as jnp
from jax import lax
from jax.experimental import pallas as pl
from jax.experimental.pallas import tpu as pltpu
from jax.experimental.pallas import tpu_sc as plsc

N = 10000
D = 128
E = 320000
NREL = 3
NC = 2
NS = 16
NW = NC * NS

HIST = 10240
HROWS = HIST // NS

E_PER_CORE = E // NC
E_PER_TILE = E // NW

AGG_CHUNK = 80
AGG_CHUNKS = E_PER_TILE // AGG_CHUNK
AGG_PAIRS = AGG_CHUNKS // 2
N_PAD = 10240
ROWS_PER_TILE = N_PAD // NS
DUMP_CHUNK = 64
DUMP_CHUNKS = ROWS_PER_TILE // DUMP_CHUNK

_mesh = plsc.VectorSubcoreMesh(core_axis_name="c", subcore_axis_name="s")


def _deg_body(s0, d0, s1, d1, s2, d2, ones, zeros, out_hbm,
              h0_sh, h1_sh, h2_sh, h3_sh, h4_sh, h5_sh,
              idx0_v, idx1_v, ones_v, zb_v, isem0, isem1):
  c = lax.axis_index("c")
  s = lax.axis_index("s")
  hists = (h0_sh, h1_sh, h2_sh, h3_sh, h4_sh, h5_sh)
  eis = (s0, d0, s1, d1, s2, d2)
  idxs = (idx0_v, idx1_v)
  isems = (isem0, isem1)
  base = c * E_PER_CORE + s * E_PER_TILE
  pltpu.sync_copy(zeros, zb_v)
  pltpu.sync_copy(ones, ones_v)
  for j in range(6):
    pltpu.sync_copy(zb_v, hists[j].at[pl.ds(s * HROWS, HROWS)])
  pltpu.async_copy(eis[0].at[pl.ds(base, E_PER_TILE)], idx0_v, isem0)
  plsc.subcore_barrier()
  for j in range(6):
    b = j % 2
    pltpu.make_async_copy(
        eis[j].at[pl.ds(base, E_PER_TILE)], idxs[b], isems[b]).wait()
    if j + 1 < 6:
      pltpu.async_copy(
          eis[j + 1].at[pl.ds(base, E_PER_TILE)], idxs[1 - b], isems[1 - b])
    pltpu.sync_copy(ones_v, hists[j].at[idxs[b]], add=True)
  plsc.subcore_barrier()
  for j in range(6):
    pltpu.sync_copy(hists[j].at[pl.ds(s * HROWS, HROWS)], zb_v)
    pltpu.sync_copy(zb_v,
                    out_hbm.at[c, 0, pl.ds(j * HIST + s * HROWS, HROWS)])


_deg_kernel = pl.kernel(
    _deg_body,
    out_type=jax.ShapeDtypeStruct((NC, 1, 6 * HIST), jnp.float32),
    mesh=_mesh,
    scratch_types=[
        pltpu.VMEM_SHARED((HIST,), jnp.float32),
        pltpu.VMEM_SHARED((HIST,), jnp.float32),
        pltpu.VMEM_SHARED((HIST,), jnp.float32),
        pltpu.VMEM_SHARED((HIST,), jnp.float32),
        pltpu.VMEM_SHARED((HIST,), jnp.float32),
        pltpu.VMEM_SHARED((HIST,), jnp.float32),
        pltpu.VMEM((E_PER_TILE,), jnp.int32),
        pltpu.VMEM((E_PER_TILE,), jnp.int32),
        pltpu.VMEM((E_PER_TILE,), jnp.float32),
        pltpu.VMEM((HROWS,), jnp.float32),
        pltpu.SemaphoreType.DMA,
        pltpu.SemaphoreType.DMA,
    ],
)


def _agg_body(h0, h1, h2, s0, d0, s1, d1, s2, d2, zeros, out0, out1, out2,
              agg_sh, sidx0, sidx1, didx0, didx1, rows0, rows1, zb_v,
              gsem0, gsem1, ssem0, ssem1,
              sisem0, sisem1, disem0, disem1):
  c = lax.axis_index("c")
  s = lax.axis_index("s")
  hs = (h0, h1, h2)
  srcs = (s0, s1, s2)
  dsts = (d0, d1, d2)
  outs = (out0, out1, out2)
  base = c * E_PER_CORE + s * E_PER_TILE

  for r in range(NREL):
    h_r, src_r, dst_r = hs[r], srcs[r], dsts[r]

    def eoff(j):
      return pl.multiple_of(base + j * AGG_CHUNK, 8)

    def load_sidx(j, sidx, sem):
      pltpu.async_copy(src_r.at[pl.ds(eoff(j), AGG_CHUNK)], sidx, sem)

    def load_didx(j, didx, sem):
      pltpu.async_copy(dst_r.at[pl.ds(eoff(j), AGG_CHUNK)], didx, sem)

    def wait_sidx(sidx, sem):
      pltpu.make_async_copy(
          src_r.at[pl.ds(eoff(0), AGG_CHUNK)], sidx, sem).wait()

    def wait_didx(didx, sem):
      pltpu.make_async_copy(
          dst_r.at[pl.ds(eoff(0), AGG_CHUNK)], didx, sem).wait()

    def issue_gather(sidx, rows, sem):
      pltpu.async_copy(h_r.at[sidx], rows, sem)

    def wait_gather(sidx, rows, sem):
      pltpu.make_async_copy(h_r.at[sidx], rows, sem).wait()

    def issue_scatter(didx, rows, sem):
      pltpu.async_copy(rows, agg_sh.at[didx], sem, add=True)

    def wait_scatter(didx, rows, sem):
      pltpu.make_async_copy(rows, agg_sh.at[didx], sem).wait()

    pltpu.sync_copy(zeros, zb_v)
    for z in range(DUMP_CHUNKS):
      pltpu.sync_copy(
          zb_v, agg_sh.at[pl.ds(s * ROWS_PER_TILE + z * DUMP_CHUNK,
                                DUMP_CHUNK)])
    load_sidx(0, sidx0, sisem0)
    load_didx(0, didx0, disem0)
    load_sidx(1, sidx1, sisem1)
    load_didx(1, didx1, disem1)
    plsc.subcore_barrier()
    wait_sidx(sidx0, sisem0)
    issue_gather(sidx0, rows0, gsem0)

    def pair(k2, carry):
      j = 2 * k2
      wait_sidx(sidx1, sisem1)
      issue_gather(sidx1, rows1, gsem1)
      wait_gather(sidx0, rows0, gsem0)

      @pl.when(j + 2 < AGG_CHUNKS)
      def _():
        load_sidx(j + 2, sidx0, sisem0)
      wait_didx(didx0, disem0)
      issue_scatter(didx0, rows0, ssem0)
      wait_scatter(didx0, rows0, ssem0)

      @pl.when(j + 2 < AGG_CHUNKS)
      def _():
        load_didx(j + 2, didx0, disem0)
        wait_sidx(sidx0, sisem0)
        issue_gather(sidx0, rows0, gsem0)
      wait_gather(sidx1, rows1, gsem1)

      @pl.when(j + 3 < AGG_CHUNKS)
      def _():
        load_sidx(j + 3, sidx1, sisem1)
      wait_didx(didx1, disem1)
      issue_scatter(didx1, rows1, ssem1)
      wait_scatter(didx1, rows1, ssem1)

      @pl.when(j + 3 < AGG_CHUNKS)
      def _():
        load_didx(j + 3, didx1, disem1)
      return carry

    lax.fori_loop(0, AGG_PAIRS, pair, 0)
    if AGG_CHUNKS % 2:
      wait_gather(sidx0, rows0, gsem0)
      wait_didx(didx0, disem0)
      issue_scatter(didx0, rows0, ssem0)
      wait_scatter(didx0, rows0, ssem0)
    plsc.subcore_barrier()
    for z in range(DUMP_CHUNKS):
      row0 = s * ROWS_PER_TILE + z * DUMP_CHUNK
      pltpu.sync_copy(agg_sh.at[pl.ds(row0, DUMP_CHUNK)], zb_v)
      pltpu.sync_copy(zb_v, outs[r].at[c, pl.ds(row0, DUMP_CHUNK)])
    plsc.subcore_barrier()


_agg_kernel = pl.kernel(
    _agg_body,
    out_type=(jax.ShapeDtypeStruct((NC, N_PAD, D), jnp.float32),) * NREL,
    mesh=_mesh,
    scratch_types=[
        pltpu.VMEM_SHARED((N_PAD, D), jnp.float32),
        pltpu.VMEM((AGG_CHUNK,), jnp.int32),
        pltpu.VMEM((AGG_CHUNK,), jnp.int32),
        pltpu.VMEM((AGG_CHUNK,), jnp.int32),
        pltpu.VMEM((AGG_CHUNK,), jnp.int32),
        pltpu.VMEM((AGG_CHUNK, D), jnp.float32),
        pltpu.VMEM((AGG_CHUNK, D), jnp.float32),
        pltpu.VMEM((DUMP_CHUNK, D), jnp.float32),
        pltpu.SemaphoreType.DMA,
        pltpu.SemaphoreType.DMA,
        pltpu.SemaphoreType.DMA,
        pltpu.SemaphoreType.DMA,
        pltpu.SemaphoreType.DMA,
        pltpu.SemaphoreType.DMA,
        pltpu.SemaphoreType.DMA,
        pltpu.SemaphoreType.DMA,
    ],
)


def _norm(deg):
  return jnp.where(deg > 0, lax.rsqrt(jnp.maximum(deg, 1e-12)), 0.0)


def _h_body(x_ref, degs_ref, h_ref):
  for r in range(NREL):
    deg = degs_ref[0, :N, 2 * r:2 * r + 1] + degs_ref[1, :N, 2 * r:2 * r + 1]
    h_ref[r] = x_ref[...] * _norm(deg)


def _h_kernel(x, degs):
  return pl.pallas_call(
      _h_body,
      out_shape=jax.ShapeDtypeStruct((NREL, N, D), jnp.float32),
  )(x, degs)


BLK = 1000


def _comb_body(degs_ref, p0_ref, p1_ref, p2_ref, w_ref, b_ref, out_ref):
  ps = (p0_ref, p1_ref, p2_ref)
  acc = jnp.zeros((BLK, D), jnp.float32)
  for r in range(NREL):
    deg = degs_ref[0, :, 2 * r + 1:2 * r + 2] + degs_ref[1, :, 2 * r + 1:2 * r + 2]
    agg = (ps[r][0] + ps[r][1]) * _norm(deg)
    acc += jnp.dot(agg, w_ref[r], preferred_element_type=jnp.float32)
  bsum = b_ref[0:1, :] + b_ref[1:2, :] + b_ref[2:3, :]
  out_ref[...] = acc * (1.0 / NREL) + bsum * (1.0 / NREL)


def _comb_kernel(degs, p0, p1, p2, W, B):
  grid = N // BLK
  part_spec = pl.BlockSpec((NC, BLK, D), lambda i: (0, i, 0))
  return pl.pallas_call(
      _comb_body,
      grid=(grid,),
      in_specs=[
          pl.BlockSpec((NC, BLK, 6), lambda i: (0, i, 0)),
          part_spec, part_spec, part_spec,
          pl.BlockSpec((NREL, D, D), lambda i: (0, 0, 0)),
          pl.BlockSpec((NREL, D), lambda i: (0, 0)),
      ],
      out_specs=pl.BlockSpec((BLK, D), lambda i: (i, 0)),
      out_shape=jax.ShapeDtypeStruct((N, D), jnp.float32),
  )(degs, p0, p1, p2, W, B)


@jax.jit
def kernel(x, edge_index_r0, edge_index_r1, edge_index_r2,
           W_r0, b_r0, W_r1, b_r1, W_r2, b_r2):
  ones = jnp.ones((E_PER_TILE,), jnp.float32)
  zeros_h = jnp.zeros((HROWS,), jnp.float32)
  zeros_a = jnp.zeros((DUMP_CHUNK, D), jnp.float32)
  s0, d0 = edge_index_r0[0], edge_index_r0[1]
  s1, d1 = edge_index_r1[0], edge_index_r1[1]
  s2, d2 = edge_index_r2[0], edge_index_r2[1]
  degs6 = _deg_kernel(s0, d0, s1, d1, s2, d2, ones, zeros_h)
  degs = jnp.swapaxes(degs6.reshape(NC, 6, HIST), 1, 2)
  h = _h_kernel(x, degs)
  p0, p1, p2 = _agg_kernel(h[0], h[1], h[2], s0, d0, s1, d1, s2, d2,
                           zeros_a)
  W = jnp.stack([W_r0, W_r1, W_r2])
  B = jnp.stack([b_r0, b_r1, b_r2])
  return _comb_kernel(degs, p0, p1, p2, W, B)
